# Initial kernel scaffold; baseline (speedup 1.0000x reference)
import jax
import jax.numpy as jnp
from jax.experimental import pallas as pl


def kernel(ui, wi, l, U, W):
    raise NotImplementedError("write your pallas kernel here")


# speedup vs baseline: 2.8592x; 2.8592x over previous
import functools

import jax
import jax.numpy as jnp
from jax import lax
from jax.experimental import pallas as pl
from jax.experimental.pallas import tpu as pltpu
from jax.experimental.pallas import tpu_sc as plsc

B = 16384
LEN = 50
D = 32
NW = 32
BPW = B // NW
HALF_BAGS = 32
HALF_ROWS = HALF_BAGS * LEN
NHALF = BPW // HALF_BAGS
IDXW = 100
GPH = HALF_ROWS // IDXW
WI_ROWS_PER_W = NHALF * GPH


def _body(ui_hbm, wi_hbm, l_hbm, U_hbm, W_hbm, out_hbm,
          widx, rows, uidx, urows, lv, outv, sem0, sem1, semu):
    cid = lax.axis_index("c")
    sid = lax.axis_index("s")
    w = sid * 2 + cid
    base = w * BPW

    pltpu.sync_copy(ui_hbm.at[pl.ds(w * 4, 4)], uidx)
    for i in range(4):
        pltpu.async_copy(U_hbm.at[uidx.at[i]], urows.at[pl.ds(i * 128, 128)], semu)
    pltpu.sync_copy(l_hbm.at[pl.ds(base, BPW)], lv)

    sems = (sem0, sem1)

    def fire(h, slot):
        pltpu.sync_copy(wi_hbm.at[pl.ds(w * WI_ROWS_PER_W + h * GPH, GPH)],
                        widx.at[slot])
        for i in range(GPH):
            pltpu.async_copy(W_hbm.at[widx.at[slot, i]],
                             rows.at[slot, pl.ds(i * IDXW, IDXW)],
                             sems[slot])

    def drain(slot):
        pltpu.make_async_copy(W_hbm.at[pl.ds(0, HALF_ROWS)], rows.at[slot],
                              sems[slot]).wait()

    fire(0, 0)
    pltpu.make_async_copy(U_hbm.at[pl.ds(0, BPW)], urows, semu).wait()

    lanes = lax.iota(jnp.int32, 16)

    gdn = lax.GatherDimensionNumbers(
        offset_dims=(), collapsed_slice_dims=(0,), start_index_map=(0,))

    def hsum(t):
        for sh in (8, 4, 2, 1):
            g = lax.gather(t, (lanes ^ sh)[:, None], gdn, slice_sizes=(1,),
                           mode=lax.GatherScatterMode.PROMISE_IN_BOUNDS)
            t = t + g
        return t

    def accum_half(h, slot):
        def bag(i, vec):
            rb = i * LEN
            a00 = rows[slot, rb, pl.ds(0, 16)]
            a01 = rows[slot, rb, pl.ds(16, 16)]
            a10 = rows[slot, rb + 1, pl.ds(0, 16)]
            a11 = rows[slot, rb + 1, pl.ds(16, 16)]
            for j in range(2, LEN, 2):
                a00 = a00 + rows[slot, rb + j, pl.ds(0, 16)]
                a01 = a01 + rows[slot, rb + j, pl.ds(16, 16)]
                a10 = a10 + rows[slot, rb + j + 1, pl.ds(0, 16)]
                a11 = a11 + rows[slot, rb + j + 1, pl.ds(16, 16)]
            s0 = a00 + a10
            s1 = a01 + a11
            gi = h * HALF_BAGS + i
            u0 = urows[gi, pl.ds(0, 16)]
            u1 = urows[gi, pl.ds(16, 16)]
            s = hsum(s0 * u0 + s1 * u1)
            lane = i & 15
            vec = jnp.where(lanes == lane, s, vec)

            @pl.when(lane == 15)
            def _():
                start = pl.multiple_of(gi - 15, 16)
                outv[pl.ds(start, 16)] = vec

            return vec
        lax.fori_loop(0, HALF_BAGS, bag, jnp.zeros((16,), jnp.float32))

    def step(it, carry):
        for b2 in range(2):
            h = it * 2 + b2

            @pl.when(h + 1 < NHALF)
            def _():
                fire(h + 1, 1 - b2)

            drain(b2)
            accum_half(h, b2)
        return carry

    lax.fori_loop(0, NHALF // 2, step, 0)

    def divloop(k, carry):
        sl = pl.ds(k * 16, 16)
        outv[sl] = outv[sl] / lv[sl].astype(jnp.float32)
        return carry

    lax.fori_loop(0, BPW // 16, divloop, 0)
    pltpu.sync_copy(outv, out_hbm.at[pl.ds(base, BPW)])


def kernel(ui, wi, l, U, W):
    ui2 = ui.astype(jnp.int32).reshape(B // 128, 128)
    wi2 = wi.astype(jnp.int32).reshape(B * LEN // IDXW, IDXW)
    l32 = l.astype(jnp.int32)
    mesh = plsc.VectorSubcoreMesh(core_axis_name="c", subcore_axis_name="s")
    run = pl.kernel(
        _body,
        mesh=mesh,
        compiler_params=pltpu.CompilerParams(use_tc_tiling_on_sc=False),
        out_type=jax.ShapeDtypeStruct((B,), jnp.float32),
        scratch_types=[
            pltpu.VMEM((2, GPH, IDXW), jnp.int32),
            pltpu.VMEM((2, HALF_ROWS, D), jnp.float32),
            pltpu.VMEM((4, 128), jnp.int32),
            pltpu.VMEM((BPW, D), jnp.float32),
            pltpu.VMEM((BPW,), jnp.int32),
            pltpu.VMEM((BPW,), jnp.float32),
            pltpu.SemaphoreType.DMA,
            pltpu.SemaphoreType.DMA,
            pltpu.SemaphoreType.DMA,
        ],
    )
    return run(ui2, wi2, l32, U, W)
